# baseline (speedup 1.0000x reference)
import jax
import jax.numpy as jnp
from jax import lax
from jax.experimental import pallas as pl
from jax.experimental.pallas import tpu as pltpu
from jax.experimental.pallas import tpu_sc as plsc

NODE_NUM = 50000
DIM = 128
K1 = 3
M = 2
NPAIR = 2 * K1
PWD = NPAIR * NPAIR
BATCH = 4096

_NC = 2
_NS = 16
_NW = _NC * _NS
_BC = BATCH // 2
_PER_W = (2 * _BC) // _NW
_CH = 128
_LANES = 16
_NTASK = K1 * (_PER_W // _CH)


def _vgather(x, idx):
    return lax.gather(
        x, idx[:, None],
        lax.GatherDimensionNumbers(offset_dims=(), collapsed_slice_dims=(0,),
                                   start_index_map=(0,)),
        (1,), mode=lax.GatherScatterMode.PROMISE_IN_BOUNDS)


def _sc_fused_gather(ids_hbm, rp_hbm, lam_hbm, out_hbm,
                     idx_v, lam_v, idx0, idx1, bb,
                     gsem, asem, osem):
    wid = lax.axis_index("s") * _NC + lax.axis_index("c")
    base = wid * _PER_W
    half = base // _BC
    brow = base - half * _BC

    pltpu.sync_copy(ids_hbm.at[pl.ds(base, _PER_W)], idx_v)

    pltpu.sync_copy(lam_hbm, lam_v.at[pl.ds(0, 6)])
    lv = lam_v[...]
    rks = []
    for k in range(K1):
        l0 = _vgather(lv, jnp.full((16,), 2 * k, jnp.int32))
        l1 = _vgather(lv, jnp.full((16,), 2 * k + 1, jnp.int32))
        rks.append(jnp.exp(l1 - l0))

    for t in range(_NTASK):
        for tt in range(_CH // _LANES):
            sld = pl.ds(tt * _LANES, _LANES)
            v = idx_v[sld]
            idx0[t, sld] = v + (t * NODE_NUM)
            idx1[t, sld] = v + ((K1 + t) * NODE_NUM)
    hg = [pltpu.async_copy(rp_hbm.at[idx1.at[t]], bb.at[t], gsem)
          for t in range(_NTASK)]
    ha = [None] * _NTASK
    for t in range(_NTASK):
        hg[t].wait()
        rk = rks[t]

        @plsc.parallel_loop(0, _CH, 1, unroll=2)
        def _scale_row(c, t=t, rk=rk):
            for l in range(DIM // _LANES):
                sl = pl.ds(l * _LANES, _LANES)
                bb[t, c, sl] = bb[t, c, sl] * rk
        ha[t] = pltpu.async_copy(rp_hbm.at[idx0.at[t]], bb.at[t], asem,
                                 add=True)
    ho = [None] * _NTASK
    for t in range(_NTASK):
        ha[t].wait()
        ho[t] = pltpu.async_copy(
            bb.at[t], out_hbm.at[half * K1 + t, pl.ds(brow, _CH)], osem)
    for t in range(_NTASK):
        ho[t].wait()


_sc_gather_call = pl.kernel(
    _sc_fused_gather,
    out_type=jax.ShapeDtypeStruct((NPAIR, _BC, DIM), jnp.float32),
    mesh=plsc.VectorSubcoreMesh(core_axis_name="c", subcore_axis_name="s"),
    scratch_types=[
        pltpu.VMEM((_PER_W,), jnp.int32),
        pltpu.VMEM((_LANES,), jnp.float32),
        pltpu.VMEM((_NTASK, _CH), jnp.int32),
        pltpu.VMEM((_NTASK, _CH), jnp.int32),
        pltpu.VMEM((_NTASK, _CH, DIM), jnp.float32),
        pltpu.SemaphoreType.DMA,
        pltpu.SemaphoreType.DMA,
        pltpu.SemaphoreType.DMA,
    ],
)


_BBLK = 512


def _tc_gram_mlp(rp_ref, lam_ref, w1_ref, b1_ref, w2_ref, b2_ref, out_ref):
    e = jnp.exp(lam_ref[...])
    w0 = []
    for k in range(K1):
        e0 = e[:, 2 * k:2 * k + 1]
        e1 = e[:, 2 * k + 1:2 * k + 2]
        w0.append(e0 / (e0 + e1))
    w0 = w0 + w0
    rows = [rp_ref[i, :, :] for i in range(NPAIR)]
    ent = {}
    for i in range(NPAIR):
        for j in range(i, NPAIR):
            ent[(i, j)] = jnp.sum(
                rows[i] * rows[j], axis=1, keepdims=True) * (w0[i] * w0[j])
    cols = []
    for i in range(NPAIR):
        for j in range(NPAIR):
            cols.append(ent[(i, j)] if i <= j else ent[(j, i)])
    feat = jnp.concatenate(cols, axis=1)
    feat = jnp.where(feat < 0.0, 0.0, feat)
    feat = jnp.log(feat + 1.0)
    h = jnp.dot(feat, w1_ref[...], preferred_element_type=jnp.float32)
    h = jnp.maximum(h + b1_ref[...], 0.0)
    out_ref[...] = (
        jnp.dot(h, w2_ref[...], preferred_element_type=jnp.float32)
        + b2_ref[...])


def _tc_call(fused, lam2, W1, b1, W2, b2):
    nblk = _BC // _BBLK
    return pl.pallas_call(
        _tc_gram_mlp,
        grid=(nblk,),
        in_specs=[
            pl.BlockSpec((NPAIR, _BBLK, DIM), lambda i: (0, i, 0)),
            pl.BlockSpec((1, K1 * M), lambda i: (0, 0)),
            pl.BlockSpec((PWD, 4 * PWD), lambda i: (0, 0)),
            pl.BlockSpec((1, 4 * PWD), lambda i: (0, 0)),
            pl.BlockSpec((4 * PWD, PWD), lambda i: (0, 0)),
            pl.BlockSpec((1, PWD), lambda i: (0, 0)),
        ],
        out_specs=pl.BlockSpec((_BBLK, PWD), lambda i: (i, 0)),
        out_shape=jax.ShapeDtypeStruct((_BC, PWD), jnp.float32),
    )(fused, lam2, W1, b1, W2, b2)


def kernel(src_node_ids, dst_node_ids, RP, lambda_weights, W1, b1, W2, b2):
    src = src_node_ids.astype(jnp.int32)
    dst = dst_node_ids.astype(jnp.int32)
    rp_flat = RP.reshape(M * K1 * NODE_NUM, DIM)
    lam_flat = lambda_weights.reshape(K1 * M).astype(jnp.float32)
    b1r = b1.reshape(1, 4 * PWD)
    b2r = b2.reshape(1, PWD)

    ids_a = jnp.concatenate([src[:_BC], dst[:_BC]])
    ids_b = jnp.concatenate([src[_BC:], dst[_BC:]])
    lam2 = lam_flat.reshape(1, K1 * M)
    fused_a = _sc_gather_call(ids_a, rp_flat, lam_flat)
    fused_b = _sc_gather_call(ids_b, rp_flat, lam_flat)
    out_a = _tc_call(fused_a, lam2, W1, b1r, W2, b2r)
    out_b = _tc_call(fused_b, lam2, W1, b1r, W2, b2r)
    return jnp.concatenate([out_a, out_b], axis=0)


# speedup vs baseline: 1.3089x; 1.3089x over previous
import jax
import jax.numpy as jnp
from jax import lax
from jax.experimental import pallas as pl
from jax.experimental.pallas import tpu as pltpu
from jax.experimental.pallas import tpu_sc as plsc

NODE_NUM = 50000
DIM = 128
K1 = 3
M = 2
NPAIR = 2 * K1
PWD = NPAIR * NPAIR
BATCH = 4096

_NC = 2
_NS = 16
_NW = _NC * _NS
_BC = BATCH // 2
_PER_W = (2 * _BC) // _NW
_CH = 128
_LANES = 16
_NTASK = K1 * (_PER_W // _CH)


def _vgather(x, idx):
    return lax.gather(
        x, idx[:, None],
        lax.GatherDimensionNumbers(offset_dims=(), collapsed_slice_dims=(0,),
                                   start_index_map=(0,)),
        (1,), mode=lax.GatherScatterMode.PROMISE_IN_BOUNDS)


def _sc_fused_gather(ids_hbm, rp_hbm, lam_hbm, out_hbm,
                     idx_v, lam_v, idx0, idx1, bb,
                     gsem, asem, osem):
    wid = lax.axis_index("s") * _NC + lax.axis_index("c")
    base = wid * _PER_W
    half = base // _BC
    brow = base - half * _BC

    pltpu.sync_copy(ids_hbm.at[pl.ds(base, _PER_W)], idx_v)

    pltpu.sync_copy(lam_hbm, lam_v.at[pl.ds(0, 6)])
    lv = lam_v[...]
    rks = []
    for k in range(K1):
        l0 = _vgather(lv, jnp.full((16,), 2 * k, jnp.int32))
        l1 = _vgather(lv, jnp.full((16,), 2 * k + 1, jnp.int32))
        rks.append(jnp.exp(l1 - l0))

    for t in range(_NTASK):
        for tt in range(_CH // _LANES):
            sld = pl.ds(tt * _LANES, _LANES)
            v = idx_v[sld]
            idx0[t, sld] = v + (t * NODE_NUM)
            idx1[t, sld] = v + ((K1 + t) * NODE_NUM)
    hg = [pltpu.async_copy(rp_hbm.at[idx1.at[t]], bb.at[t], gsem)
          for t in range(_NTASK)]
    ha = [None] * _NTASK
    for t in range(_NTASK):
        hg[t].wait()
        rk = rks[t]

        @plsc.parallel_loop(0, _CH, 1, unroll=2)
        def _scale_row(c, t=t, rk=rk):
            for l in range(DIM // _LANES):
                sl = pl.ds(l * _LANES, _LANES)
                bb[t, c, sl] = bb[t, c, sl] * rk
        ha[t] = pltpu.async_copy(rp_hbm.at[idx0.at[t]], bb.at[t], asem,
                                 add=True)
    ho = [None] * _NTASK
    for t in range(_NTASK):
        ha[t].wait()
        ho[t] = pltpu.async_copy(
            bb.at[t], out_hbm.at[half * K1 + t, pl.ds(brow, _CH)], osem)
    for t in range(_NTASK):
        ho[t].wait()


_sc_gather_call = pl.kernel(
    _sc_fused_gather,
    out_type=jax.ShapeDtypeStruct((NPAIR, _BC, DIM), jnp.float32),
    mesh=plsc.VectorSubcoreMesh(core_axis_name="c", subcore_axis_name="s"),
    scratch_types=[
        pltpu.VMEM((_PER_W,), jnp.int32),
        pltpu.VMEM((_LANES,), jnp.float32),
        pltpu.VMEM((_NTASK, _CH), jnp.int32),
        pltpu.VMEM((_NTASK, _CH), jnp.int32),
        pltpu.VMEM((_NTASK, _CH, DIM), jnp.float32),
        pltpu.SemaphoreType.DMA,
        pltpu.SemaphoreType.DMA,
        pltpu.SemaphoreType.DMA,
    ],
)


_BBLK = 512


def _tc_gram_mlp(rp_ref, lam_ref, w1_ref, b1_ref, w2_ref, b2_ref, out_ref):
    e = jnp.exp(lam_ref[...])
    w0 = []
    for k in range(K1):
        e0 = e[:, 2 * k:2 * k + 1]
        e1 = e[:, 2 * k + 1:2 * k + 2]
        w0.append(e0 / (e0 + e1))
    w0 = w0 + w0
    rows = [rp_ref[i, :, :] * w0[i] for i in range(NPAIR)]
    ent = {}
    for i in range(NPAIR):
        for j in range(i, NPAIR):
            ent[(i, j)] = jnp.sum(rows[i] * rows[j], axis=1, keepdims=True)
    cols = []
    for i in range(NPAIR):
        for j in range(NPAIR):
            cols.append(ent[(i, j)] if i <= j else ent[(j, i)])
    feat = jnp.concatenate(cols, axis=1)
    feat = jnp.where(feat < 0.0, 0.0, feat)
    feat = jnp.log(feat + 1.0)
    h = jnp.dot(feat, w1_ref[...], preferred_element_type=jnp.float32)
    h = jnp.maximum(h + b1_ref[...], 0.0)
    out_ref[...] = (
        jnp.dot(h, w2_ref[...], preferred_element_type=jnp.float32)
        + b2_ref[...])


def _tc_call(fused, lam2, W1, b1, W2, b2):
    nblk = _BC // _BBLK
    return pl.pallas_call(
        _tc_gram_mlp,
        grid=(nblk,),
        in_specs=[
            pl.BlockSpec((NPAIR, _BBLK, DIM), lambda i: (0, i, 0)),
            pl.BlockSpec((1, K1 * M), lambda i: (0, 0)),
            pl.BlockSpec((PWD, 4 * PWD), lambda i: (0, 0)),
            pl.BlockSpec((1, 4 * PWD), lambda i: (0, 0)),
            pl.BlockSpec((4 * PWD, PWD), lambda i: (0, 0)),
            pl.BlockSpec((1, PWD), lambda i: (0, 0)),
        ],
        out_specs=pl.BlockSpec((_BBLK, PWD), lambda i: (i, 0)),
        out_shape=jax.ShapeDtypeStruct((_BC, PWD), jnp.float32),
    )(fused, lam2, W1, b1, W2, b2)


def kernel(src_node_ids, dst_node_ids, RP, lambda_weights, W1, b1, W2, b2):
    src = src_node_ids.astype(jnp.int32)
    dst = dst_node_ids.astype(jnp.int32)
    rp_flat = RP.reshape(M * K1 * NODE_NUM, DIM)
    lam_flat = lambda_weights.reshape(K1 * M).astype(jnp.float32)
    b1r = b1.reshape(1, 4 * PWD)
    b2r = b2.reshape(1, PWD)

    ids_a = jnp.concatenate([src[:_BC], dst[:_BC]])
    ids_b = jnp.concatenate([src[_BC:], dst[_BC:]])
    lam2 = lam_flat.reshape(1, K1 * M)
    fused_a = _sc_gather_call(ids_a, rp_flat, lam_flat)
    fused_b = _sc_gather_call(ids_b, rp_flat, lam_flat)
    out_a = _tc_call(fused_a, lam2, W1, b1r, W2, b2r)
    out_b = _tc_call(fused_b, lam2, W1, b1r, W2, b2r)
    return jnp.concatenate([out_a, out_b], axis=0)
